# baseline (speedup 1.0000x reference)
import functools

import jax
import jax.numpy as jnp
from jax import lax
from jax.experimental import pallas as pl
from jax.experimental.pallas import tpu as pltpu
from jax.experimental.pallas import tpu_sc as plsc

S = 16384
DIM = 2048
E = 16
K = 2
TS = 2048

NC = 2
NS = 16
NW = NC * NS
TPW = S // NW
GRP = TPW // 16


def _gate_tc_body(x_ref, w_ref, b_ref, probs_t_ref, imp_ref, imp_acc):
    step = pl.program_id(0)
    nsteps = pl.num_programs(0)

    logits = jnp.dot(x_ref[...], w_ref[...],
                     preferred_element_type=jnp.float32) + b_ref[...]
    m = jnp.max(logits, axis=-1, keepdims=True)
    ex = jnp.exp(logits - m)
    denom = jnp.sum(ex, axis=-1, keepdims=True)
    probs = ex / denom

    probs_t_ref[...] = probs.T

    imp_part = jnp.sum(probs, axis=0, keepdims=True)

    @pl.when(step == 0)
    def _init():
        imp_acc[...] = imp_part

    @pl.when(step != 0)
    def _accum():
        imp_acc[...] += imp_part

    @pl.when(step == nsteps - 1)
    def _finalize():
        imp_ref[...] = imp_acc[...]


def _probs_and_importance(x, W, b):
    b2 = b.reshape(1, E)
    return pl.pallas_call(
        _gate_tc_body,
        grid=(S // TS,),
        in_specs=[
            pl.BlockSpec((TS, DIM), lambda i: (i, 0)),
            pl.BlockSpec((DIM, E), lambda i: (0, 0)),
            pl.BlockSpec((1, E), lambda i: (0, 0)),
        ],
        out_specs=(
            pl.BlockSpec((E, TS), lambda i: (0, i)),
            pl.BlockSpec((1, E), lambda i: (0, 0)),
        ),
        out_shape=(
            jax.ShapeDtypeStruct((E, S), jnp.float32),
            jax.ShapeDtypeStruct((1, E), jnp.float32),
        ),
        scratch_shapes=[pltpu.VMEM((1, E), jnp.float32)],
        compiler_params=pltpu.CompilerParams(
            dimension_semantics=("arbitrary",),
            vmem_limit_bytes=50 * 1024 * 1024,
        ),
    )(x, W, b2)


def _topk_sc(probs_t):
    mesh = plsc.VectorSubcoreMesh(core_axis_name="c", subcore_axis_name="s",
                                  num_cores=NC)

    @functools.partial(
        pl.kernel,
        mesh=mesh,
        out_type=(
            jax.ShapeDtypeStruct((K, S), jnp.int32),
            jax.ShapeDtypeStruct((K, S), jnp.float32),
            jax.ShapeDtypeStruct((NW, E), jnp.float32),
        ),
        scratch_types=[
            pltpu.VMEM((E, TPW), jnp.float32),
            pltpu.VMEM((K, TPW), jnp.int32),
            pltpu.VMEM((K, TPW), jnp.float32),
            pltpu.VMEM((E,), jnp.float32),
        ],
        compiler_params=pltpu.CompilerParams(needs_layout_passes=False),
    )
    def sck(probs_hbm, idx_hbm, vals_hbm, hist_hbm,
            probs_v, idx_v, vals_v, hist_v):
        cid = lax.axis_index("c")
        sid = lax.axis_index("s")
        wid = cid * NS + sid
        base = wid * TPW

        pltpu.sync_copy(probs_hbm.at[:, pl.ds(base, TPW)], probs_v)

        hist_v[...] = jnp.zeros((E,), jnp.float32)
        ones_f = jnp.ones((16,), jnp.float32)

        def group(g, carry):
            t0 = g * 16
            v1 = jnp.full((16,), -jnp.inf, jnp.float32)
            v2 = jnp.full((16,), -jnp.inf, jnp.float32)
            i1 = jnp.zeros((16,), jnp.int32)
            i2 = jnp.zeros((16,), jnp.int32)
            for e in range(E):
                p = probs_v[e, pl.ds(t0, 16)]
                gt1 = p > v1
                gt2 = jnp.logical_and(jnp.logical_not(gt1), p > v2)
                i2 = jnp.where(gt1, i1, jnp.where(gt2, e, i2))
                v2 = jnp.where(gt1, v1, jnp.where(gt2, p, v2))
                i1 = jnp.where(gt1, e, i1)
                v1 = jnp.where(gt1, p, v1)
            idx_v[0, pl.ds(t0, 16)] = i1
            idx_v[1, pl.ds(t0, 16)] = i2
            vals_v[0, pl.ds(t0, 16)] = v1
            vals_v[1, pl.ds(t0, 16)] = v2
            plsc.addupdate_scatter(hist_v, [i1], ones_f)
            return carry

        lax.fori_loop(0, GRP, group, 0, unroll=2)

        pltpu.sync_copy(idx_v, idx_hbm.at[:, pl.ds(base, TPW)])
        pltpu.sync_copy(vals_v, vals_hbm.at[:, pl.ds(base, TPW)])
        pltpu.sync_copy(hist_v, hist_hbm.at[wid])

    return sck(probs_t)


def _aux_body(hists_ref, imp_ref, aux_ref):
    load_raw = jnp.sum(hists_ref[...], axis=0, keepdims=True)
    dot = jnp.sum(load_raw * imp_ref[...], axis=-1, keepdims=True)
    aux_ref[...] = dot * jnp.float32(E) / jnp.float32(S) / jnp.float32(S)


def _aux_tc(hists, imp_raw):
    return pl.pallas_call(
        _aux_body,
        out_shape=jax.ShapeDtypeStruct((1, 1), jnp.float32),
    )(hists, imp_raw)


def kernel(x, W, b):
    probs_t, imp_raw = _probs_and_importance(x, W, b)
    idx_t, vals_t, hists = _topk_sc(probs_t)
    aux = _aux_tc(hists, imp_raw)
    return (idx_t.T, vals_t.T, aux[0, 0])


# speedup vs baseline: 1.0051x; 1.0051x over previous
import functools

import jax
import jax.numpy as jnp
from jax import lax
from jax.experimental import pallas as pl
from jax.experimental.pallas import tpu as pltpu
from jax.experimental.pallas import tpu_sc as plsc

S = 16384
DIM = 2048
E = 16
K = 2
TS = 2048

NC = 2
NS = 16
NW = NC * NS
TPW = S // NW
GRP = TPW // 16


def _gate_tc_body(x_ref, w_ref, b_ref, probs_t_ref, imp_ref, imp_acc):
    step = pl.program_id(0)
    nsteps = pl.num_programs(0)

    logits = jnp.dot(x_ref[...], w_ref[...],
                     preferred_element_type=jnp.float32) + b_ref[...]
    m = jnp.max(logits, axis=-1, keepdims=True)
    ex = jnp.exp(logits - m)
    denom = jnp.sum(ex, axis=-1, keepdims=True)
    probs = ex / denom

    probs_t_ref[...] = probs.T

    imp_part = jnp.sum(probs, axis=0, keepdims=True)

    @pl.when(step == 0)
    def _init():
        imp_acc[...] = imp_part

    @pl.when(step != 0)
    def _accum():
        imp_acc[...] += imp_part

    @pl.when(step == nsteps - 1)
    def _finalize():
        imp_ref[...] = imp_acc[...]


def _probs_and_importance(x, W, b):
    b2 = b.reshape(1, E)
    return pl.pallas_call(
        _gate_tc_body,
        grid=(S // TS,),
        in_specs=[
            pl.BlockSpec((TS, DIM), lambda i: (i, 0)),
            pl.BlockSpec((DIM, E), lambda i: (0, 0)),
            pl.BlockSpec((1, E), lambda i: (0, 0)),
        ],
        out_specs=(
            pl.BlockSpec((E, TS), lambda i: (0, i)),
            pl.BlockSpec((1, E), lambda i: (0, 0)),
        ),
        out_shape=(
            jax.ShapeDtypeStruct((E, S), jnp.float32),
            jax.ShapeDtypeStruct((1, E), jnp.float32),
        ),
        scratch_shapes=[pltpu.VMEM((1, E), jnp.float32)],
        compiler_params=pltpu.CompilerParams(
            dimension_semantics=("arbitrary",),
            vmem_limit_bytes=50 * 1024 * 1024,
        ),
    )(x, W, b2)


def _topk_sc(probs_t):
    mesh = plsc.VectorSubcoreMesh(core_axis_name="c", subcore_axis_name="s",
                                  num_cores=NC)

    @functools.partial(
        pl.kernel,
        mesh=mesh,
        out_type=(
            jax.ShapeDtypeStruct((K, S), jnp.int32),
            jax.ShapeDtypeStruct((K, S), jnp.float32),
            jax.ShapeDtypeStruct((NW, E), jnp.float32),
        ),
        scratch_types=[
            pltpu.VMEM((E, TPW), jnp.float32),
            pltpu.VMEM((K, TPW), jnp.int32),
            pltpu.VMEM((K, TPW), jnp.float32),
            pltpu.VMEM((E,), jnp.float32),
        ],
        compiler_params=pltpu.CompilerParams(needs_layout_passes=False),
    )
    def sck(probs_hbm, idx_hbm, vals_hbm, hist_hbm,
            probs_v, idx_v, vals_v, hist_v):
        cid = lax.axis_index("c")
        sid = lax.axis_index("s")
        wid = cid * NS + sid
        base = wid * TPW

        pltpu.sync_copy(probs_hbm.at[:, pl.ds(base, TPW)], probs_v)

        hist_v[...] = jnp.zeros((E,), jnp.float32)
        ones_f = jnp.ones((16,), jnp.float32)

        def group(g, carry):
            t0 = g * 16
            v1 = jnp.full((16,), -jnp.inf, jnp.float32)
            v2 = jnp.full((16,), -jnp.inf, jnp.float32)
            i1 = jnp.zeros((16,), jnp.int32)
            i2 = jnp.zeros((16,), jnp.int32)
            for e in range(E):
                p = probs_v[e, pl.ds(t0, 16)]
                gt1 = p > v1
                gt2 = jnp.logical_and(jnp.logical_not(gt1), p > v2)
                i2 = jnp.where(gt1, i1, jnp.where(gt2, e, i2))
                v2 = jnp.where(gt1, v1, jnp.where(gt2, p, v2))
                i1 = jnp.where(gt1, e, i1)
                v1 = jnp.where(gt1, p, v1)
            idx_v[0, pl.ds(t0, 16)] = i1
            idx_v[1, pl.ds(t0, 16)] = i2
            vals_v[0, pl.ds(t0, 16)] = v1
            vals_v[1, pl.ds(t0, 16)] = v2
            plsc.addupdate_scatter(hist_v, [i1], ones_f)
            return carry

        lax.fori_loop(0, GRP, group, 0)

        pltpu.sync_copy(idx_v, idx_hbm.at[:, pl.ds(base, TPW)])
        pltpu.sync_copy(vals_v, vals_hbm.at[:, pl.ds(base, TPW)])
        pltpu.sync_copy(hist_v, hist_hbm.at[wid])

    return sck(probs_t)


def _aux_body(hists_ref, imp_ref, aux_ref):
    load_raw = jnp.sum(hists_ref[...], axis=0, keepdims=True)
    dot = jnp.sum(load_raw * imp_ref[...], axis=-1, keepdims=True)
    aux_ref[...] = dot * jnp.float32(E) / jnp.float32(S) / jnp.float32(S)


def _aux_tc(hists, imp_raw):
    return pl.pallas_call(
        _aux_body,
        out_shape=jax.ShapeDtypeStruct((1, 1), jnp.float32),
    )(hists, imp_raw)


def kernel(x, W, b):
    probs_t, imp_raw = _probs_and_importance(x, W, b)
    idx_t, vals_t, hists = _topk_sc(probs_t)
    aux = _aux_tc(hists, imp_raw)
    return (idx_t.T, vals_t.T, aux[0, 0])
